# Initial kernel scaffold; baseline (speedup 1.0000x reference)
import functools

import jax
import jax.numpy as jnp
from jax import lax
from jax.experimental import pallas as pl
from jax.experimental.pallas import tpu as pltpu
from jax.experimental.pallas import tpu_sc as plsc

_N = 10000
_C = 128
_E = 320000
_NC = 2
_NS = 16
_NW = _NC * _NS
_EPT = _E // _NW
_K = 80
_CHUNKS = _EPT // _K
_RPT = _N // _NS


def _segsum_body(x_hbm, src_hbm, dst_hbm, w_hbm, out_hbm,
                 acc, z16, srcb, dstb, wb, msg, sem):
    c = lax.axis_index("c")
    s = lax.axis_index("s")
    tid = c * _NS + s

    zero = jnp.zeros((16,), jnp.float32)
    for r in range(16):
        for j in range(8):
            z16[r, pl.ds(j * 16, 16)] = zero

    def _zero_chunk(i, carry):
        pltpu.sync_copy(z16, acc.at[pl.ds(s * _RPT + i * 16, 16)])
        return carry

    lax.fori_loop(0, _RPT // 16, _zero_chunk, 0)
    pltpu.sync_copy(z16.at[pl.ds(0, 1)], acc.at[pl.ds(s * _RPT + _RPT - 1, 1)])
    plsc.subcore_barrier()

    ebase = tid * _EPT

    def _edge_chunk(k, carry):
        off = ebase + k * _K
        pltpu.sync_copy(src_hbm.at[pl.ds(off, _K)], srcb)
        pltpu.sync_copy(dst_hbm.at[pl.ds(off, _K)], dstb)
        pltpu.sync_copy(w_hbm.at[pl.ds(off, _K)], wb)
        pltpu.async_copy(x_hbm.at[srcb], msg, sem).wait()
        for e in range(_K):
            wsp = plsc.load_gather(wb, [jnp.full((16,), e, jnp.int32)])
            for j in range(8):
                msg[e, pl.ds(j * 16, 16)] = msg[e, pl.ds(j * 16, 16)] * wsp
        pltpu.sync_copy(msg, acc.at[dstb], add=True)
        return carry

    lax.fori_loop(0, _CHUNKS, _edge_chunk, 0)
    plsc.subcore_barrier()
    pltpu.sync_copy(acc.at[pl.ds(s * _RPT, _RPT)],
                    out_hbm.at[c, pl.ds(s * _RPT, _RPT)])


def _segsum_sc(x, src, dst, w):
    mesh = plsc.VectorSubcoreMesh(core_axis_name="c", subcore_axis_name="s")
    return pl.kernel(
        _segsum_body,
        out_type=jax.ShapeDtypeStruct((_NC, _N, _C), jnp.float32),
        mesh=mesh,
        scratch_types=[
            pltpu.VMEM_SHARED((_N, _C), jnp.float32),
            pltpu.VMEM((16, _C), jnp.float32),
            pltpu.VMEM((_K,), jnp.int32),
            pltpu.VMEM((_K,), jnp.int32),
            pltpu.VMEM((_K,), jnp.float32),
            pltpu.VMEM((_K, _C), jnp.float32),
            pltpu.SemaphoreType.DMA,
        ],
    )(x, src, dst, w)


_BR = 1000


def _dense_body(lift, x_ref, p0_ref, p1_ref, w_ref, b_ref, o_ref):
    g = p0_ref[...] + p1_ref[...]
    if lift:
        nrm = jnp.sqrt(jnp.sum(g * g, axis=1, keepdims=True))
        g = g / jnp.maximum(nrm, 1e-12)
    t = (jnp.dot(x_ref[...], w_ref[0:_C, :], preferred_element_type=jnp.float32)
         + jnp.dot(g, w_ref[_C:2 * _C, :], preferred_element_type=jnp.float32)
         + b_ref[...])
    if lift:
        nrm = jnp.sqrt(jnp.sum(t * t, axis=1, keepdims=True))
        t = t / jnp.maximum(nrm, 1e-12)
    else:
        t = jnp.tanh(t)
    o_ref[...] = t


def _dense_tc(x, p, wT, b2, lift):
    return pl.pallas_call(
        functools.partial(_dense_body, lift),
        grid=(_N // _BR,),
        in_specs=[
            pl.BlockSpec((_BR, _C), lambda i: (i, 0)),
            pl.BlockSpec((_BR, _C), lambda i: (i, 0)),
            pl.BlockSpec((_BR, _C), lambda i: (i, 0)),
            pl.BlockSpec((2 * _C, _C), lambda i: (0, 0)),
            pl.BlockSpec((1, _C), lambda i: (0, 0)),
        ],
        out_specs=pl.BlockSpec((_BR, _C), lambda i: (i, 0)),
        out_shape=jax.ShapeDtypeStruct((_N, _C), jnp.float32),
    )(x, p[0], p[1], wT, b2)


def kernel(x, edge_index, edge_weight, W_lift, b_lift, W_proj, b_proj):
    src = edge_index[0].astype(jnp.int32)
    dst = edge_index[1].astype(jnp.int32)
    w = edge_weight.astype(jnp.float32)
    for i in range(W_lift.shape[0]):
        p = _segsum_sc(x, src, dst, w)
        x = _dense_tc(x, p, W_lift[i].T, b_lift[i][None, :], lift=True)
    for i in range(W_proj.shape[0]):
        p = _segsum_sc(x, src, dst, w)
        x = _dense_tc(x, p, W_proj[i].T, b_proj[i][None, :], lift=False)
    return x


# speedup vs baseline: 3.6021x; 3.6021x over previous
import functools

import jax
import jax.numpy as jnp
from jax import lax
from jax.experimental import pallas as pl
from jax.experimental.pallas import tpu as pltpu
from jax.experimental.pallas import tpu_sc as plsc

_N = 10000
_C = 128
_E = 320000
_NC = 2
_NS = 16
_NW = _NC * _NS
_EPT = _E // _NW
_K = 80
_CHUNKS = _EPT // _K
_RPT = 632
_NP = _RPT * _NS


def _segsum_body(x_hbm, src_hbm, dst_hbm, w_hbm, out_hbm,
                 acc, z16, srcb, dstb, wb, msg, sem):
    c = lax.axis_index("c")
    s = lax.axis_index("s")
    tid = c * _NS + s

    zero = jnp.zeros((16,), jnp.float32)
    for r in range(16):
        for j in range(8):
            z16[r, pl.ds(j * 16, 16)] = zero

    def _zero_chunk(i, carry):
        pltpu.sync_copy(z16, acc.at[pl.ds(s * _RPT + i * 16, 16)])
        return carry

    lax.fori_loop(0, _RPT // 16, _zero_chunk, 0)
    pltpu.sync_copy(z16.at[pl.ds(0, 8)], acc.at[pl.ds(s * _RPT + _RPT - 8, 8)])
    plsc.subcore_barrier()

    ebase = tid * _EPT

    def _edge_chunk(k, carry):
        off = ebase + k * _K
        pltpu.sync_copy(src_hbm.at[pl.ds(off, _K)], srcb)
        pltpu.sync_copy(dst_hbm.at[pl.ds(off, _K)], dstb)
        pltpu.sync_copy(w_hbm.at[pl.ds(off, _K)], wb)
        pltpu.async_copy(x_hbm.at[srcb], msg, sem).wait()
        for t in range(_K // 16):
            w16 = wb[pl.ds(t * 16, 16)]
            for u in range(16):
                e = t * 16 + u
                wsp = jnp.full((16,), w16[u], jnp.float32)
                for j in range(8):
                    msg[e, pl.ds(j * 16, 16)] = msg[e, pl.ds(j * 16, 16)] * wsp
        pltpu.sync_copy(msg, acc.at[dstb], add=True)
        return carry

    lax.fori_loop(0, _CHUNKS, _edge_chunk, 0)
    plsc.subcore_barrier()
    pltpu.sync_copy(acc.at[pl.ds(s * _RPT, _RPT)],
                    out_hbm.at[c, pl.ds(s * _RPT, _RPT)])


def _segsum_sc(x, src, dst, w):
    mesh = plsc.VectorSubcoreMesh(core_axis_name="c", subcore_axis_name="s")
    return pl.kernel(
        _segsum_body,
        out_type=jax.ShapeDtypeStruct((_NC, _NP, _C), jnp.float32),
        mesh=mesh,
        scratch_types=[
            pltpu.VMEM_SHARED((_NP, _C), jnp.float32),
            pltpu.VMEM((16, _C), jnp.float32),
            pltpu.VMEM((_K,), jnp.int32),
            pltpu.VMEM((_K,), jnp.int32),
            pltpu.VMEM((_K,), jnp.float32),
            pltpu.VMEM((_K, _C), jnp.float32),
            pltpu.SemaphoreType.DMA,
        ],
    )(x, src, dst, w)


_BR = 1000


def _dense_body(lift, x_ref, p0_ref, p1_ref, w_ref, b_ref, o_ref):
    g = p0_ref[...] + p1_ref[...]
    if lift:
        nrm = jnp.sqrt(jnp.sum(g * g, axis=1, keepdims=True))
        g = g / jnp.maximum(nrm, 1e-12)
    h = jnp.concatenate([x_ref[...], g], axis=1)
    t = jnp.dot(h, w_ref[...], preferred_element_type=jnp.float32) + b_ref[...]
    if lift:
        nrm = jnp.sqrt(jnp.sum(t * t, axis=1, keepdims=True))
        t = t / jnp.maximum(nrm, 1e-12)
    else:
        t = jnp.tanh(t)
    o_ref[...] = t


def _dense_tc(x, p, wT, b2, lift):
    return pl.pallas_call(
        functools.partial(_dense_body, lift),
        grid=(_N // _BR,),
        in_specs=[
            pl.BlockSpec((_BR, _C), lambda i: (i, 0)),
            pl.BlockSpec((_BR, _C), lambda i: (i, 0)),
            pl.BlockSpec((_BR, _C), lambda i: (i, 0)),
            pl.BlockSpec((2 * _C, _C), lambda i: (0, 0)),
            pl.BlockSpec((1, _C), lambda i: (0, 0)),
        ],
        out_specs=pl.BlockSpec((_BR, _C), lambda i: (i, 0)),
        out_shape=jax.ShapeDtypeStruct((_N, _C), jnp.float32),
    )(x, p[0], p[1], wT, b2)


def kernel(x, edge_index, edge_weight, W_lift, b_lift, W_proj, b_proj):
    src = edge_index[0].astype(jnp.int32)
    dst = edge_index[1].astype(jnp.int32)
    w = edge_weight.astype(jnp.float32)
    for i in range(W_lift.shape[0]):
        p = _segsum_sc(x, src, dst, w)
        x = _dense_tc(x, p, W_lift[i].T, b_lift[i][None, :], lift=True)
    for i in range(W_proj.shape[0]):
        p = _segsum_sc(x, src, dst, w)
        x = _dense_tc(x, p, W_proj[i].T, b_proj[i][None, :], lift=False)
    return x


# speedup vs baseline: 5.9717x; 1.6579x over previous
import functools

import jax
import jax.numpy as jnp
from jax import lax
from jax.experimental import pallas as pl
from jax.experimental.pallas import tpu as pltpu
from jax.experimental.pallas import tpu_sc as plsc

_N = 10000
_C = 128
_E = 320000
_NC = 2
_NS = 16
_NW = _NC * _NS
_EPT = _E // _NW
_K = 48
_CHUNKS = 210
_EPTP = _CHUNKS * _K
_NB = 3
_ROUNDS = _CHUNKS // _NB
_RPT = 632
_NP = _RPT * _NS


def _segsum_body(x_hbm, src_hbm, dst_hbm, w_hbm, out_hbm,
                 acc, src1, dst1, w1,
                 msg0, msg1, msg2,
                 sg0, sg1, sg2, ss0, ss1, ss2):
    c = lax.axis_index("c")
    s = lax.axis_index("s")
    tid = c * _NS + s
    msgs = (msg0, msg1, msg2)
    sgs = (sg0, sg1, sg2)
    sss = (ss0, ss1, ss2)

    pltpu.sync_copy(src_hbm.at[tid], src1)
    pltpu.sync_copy(dst_hbm.at[tid], dst1)
    pltpu.sync_copy(w_hbm.at[tid], w1)

    zero = jnp.zeros((16,), jnp.float32)
    for r in range(8):
        for j in range(8):
            msg0[r, pl.ds(j * 16, 16)] = zero

    def _zero_chunk(i, carry):
        pltpu.sync_copy(msg0.at[pl.ds(0, 8)],
                        acc.at[pl.ds(s * _RPT + i * 8, 8)])
        return carry

    lax.fori_loop(0, _RPT // 8, _zero_chunk, 0)
    plsc.subcore_barrier()

    def _gather_idx(k):
        return src1.at[pl.ds(k * _K, _K)]

    pltpu.async_copy(x_hbm.at[_gather_idx(0)], msg0, sg0)
    pltpu.async_copy(x_hbm.at[_gather_idx(1)], msg1, sg1)

    def _drain_one(bm):
        d16 = dst1[pl.ds(0, 16)]
        pltpu.make_async_copy(
            msgs[bm].at[pl.ds(0, 16)], acc.at[d16], sss[bm]).wait()

    def _round(i, carry):
        for b in range(_NB):
            k = _NB * i + b
            mb = msgs[b]
            pltpu.make_async_copy(x_hbm.at[_gather_idx(k)], mb, sgs[b]).wait()

            def _group(t, carry2, mb=mb, k=k, b=b):
                base = k * _K + t * 16
                w16 = w1[pl.ds(base, 16)]
                d16 = dst1[pl.ds(base, 16)]
                for u in range(16):
                    row = t * 16 + u
                    wsp = jnp.full((16,), w16[u], jnp.float32)
                    for j in range(8):
                        mb[row, pl.ds(j * 16, 16)] = (
                            mb[row, pl.ds(j * 16, 16)] * wsp)
                pltpu.async_copy(mb.at[pl.ds(t * 16, 16)], acc.at[d16],
                                 sss[b], add=True)
                return carry2

            lax.fori_loop(0, _K // 16, _group, 0)

            bm = (b + 2) % _NB

            def _pf(k=k, bm=bm, drain=True):
                if drain:
                    for _ in range(_K // 16):
                        _drain_one(bm)
                pltpu.async_copy(x_hbm.at[_gather_idx(k + 2)], msgs[bm],
                                 sgs[bm])

            if b == 0:
                pl.when(i >= 1)(lambda k=k, bm=bm: _pf(k, bm, True))
                pl.when(i == 0)(lambda k=k, bm=bm: _pf(k, bm, False))
            else:
                pl.when(i < _ROUNDS - 1)(lambda k=k, bm=bm: _pf(k, bm, True))
        return carry

    lax.fori_loop(0, _ROUNDS, _round, 0)

    for b in range(_NB):
        for _ in range(_K // 16):
            _drain_one(b)
    plsc.subcore_barrier()
    pltpu.sync_copy(acc.at[pl.ds(s * _RPT, _RPT)],
                    out_hbm.at[c, pl.ds(s * _RPT, _RPT)])


def _segsum_sc(x, src2, dst2, w2):
    mesh = plsc.VectorSubcoreMesh(core_axis_name="c", subcore_axis_name="s")
    return pl.kernel(
        _segsum_body,
        out_type=jax.ShapeDtypeStruct((_NC, _NP, _C), jnp.float32),
        mesh=mesh,
        scratch_types=[
            pltpu.VMEM_SHARED((_NP, _C), jnp.float32),
            pltpu.VMEM((_EPTP,), jnp.int32),
            pltpu.VMEM((_EPTP,), jnp.int32),
            pltpu.VMEM((_EPTP,), jnp.float32),
            pltpu.VMEM((_K, _C), jnp.float32),
            pltpu.VMEM((_K, _C), jnp.float32),
            pltpu.VMEM((_K, _C), jnp.float32),
            pltpu.SemaphoreType.DMA,
            pltpu.SemaphoreType.DMA,
            pltpu.SemaphoreType.DMA,
            pltpu.SemaphoreType.DMA,
            pltpu.SemaphoreType.DMA,
            pltpu.SemaphoreType.DMA,
        ],
    )(x, src2, dst2, w2)


_BR = 1000


def _dense_body(lift, x_ref, p0_ref, p1_ref, w_ref, b_ref, o_ref):
    g = p0_ref[...] + p1_ref[...]
    if lift:
        nrm = jnp.sqrt(jnp.sum(g * g, axis=1, keepdims=True))
        g = g / jnp.maximum(nrm, 1e-12)
    h = jnp.concatenate([x_ref[...], g], axis=1)
    t = jnp.dot(h, w_ref[...], preferred_element_type=jnp.float32) + b_ref[...]
    if lift:
        nrm = jnp.sqrt(jnp.sum(t * t, axis=1, keepdims=True))
        t = t / jnp.maximum(nrm, 1e-12)
    else:
        t = jnp.tanh(t)
    o_ref[...] = t


def _dense_tc(x, p, wT, b2, lift):
    return pl.pallas_call(
        functools.partial(_dense_body, lift),
        grid=(_N // _BR,),
        in_specs=[
            pl.BlockSpec((_BR, _C), lambda i: (i, 0)),
            pl.BlockSpec((_BR, _C), lambda i: (i, 0)),
            pl.BlockSpec((_BR, _C), lambda i: (i, 0)),
            pl.BlockSpec((2 * _C, _C), lambda i: (0, 0)),
            pl.BlockSpec((1, _C), lambda i: (0, 0)),
        ],
        out_specs=pl.BlockSpec((_BR, _C), lambda i: (i, 0)),
        out_shape=jax.ShapeDtypeStruct((_N, _C), jnp.float32),
    )(x, p[0], p[1], wT, b2)


def _pack_edges(v, dtype):
    v = v.astype(dtype).reshape(_NW, _EPT)
    return jnp.pad(v, ((0, 0), (0, _EPTP - _EPT)))


def kernel(x, edge_index, edge_weight, W_lift, b_lift, W_proj, b_proj):
    src2 = _pack_edges(edge_index[0], jnp.int32)
    dst2 = _pack_edges(edge_index[1], jnp.int32)
    w2 = _pack_edges(edge_weight, jnp.float32)
    for i in range(W_lift.shape[0]):
        p = _segsum_sc(x, src2, dst2, w2)
        x = _dense_tc(x, p, W_lift[i].T, b_lift[i][None, :], lift=True)
    for i in range(W_proj.shape[0]):
        p = _segsum_sc(x, src2, dst2, w2)
        x = _dense_tc(x, p, W_proj[i].T, b_proj[i][None, :], lift=False)
    return x
